# baseline (speedup 1.0000x reference)
import functools

import jax
import jax.numpy as jnp
from jax.experimental import pallas as pl
from jax.experimental.pallas import tpu as pltpu

H = 128
W = 128
FCHUNK = 512


def _transform_vertices(v):
    h, w = H, W
    vx = -v[..., 0]
    vy = -v[..., 1]
    vz = v[..., 2]
    vx = vx * w / 2 + w / 2
    vy = vy * h / 2 + h / 2
    vx = w - 1 - vx
    vy = h - 1 - vy
    vx = -1 + (2 * vx + 1) / w
    vy = -1 + (2 * vy + 1) / h
    vx = vx * w / 2 + w / 2
    vy = vy * h / 2 + h / 2
    vz = vz * w / 2
    return vx, vy, vz


def _face_coeffs(vertices, faces):
    vx, vy, vz = _transform_vertices(vertices.astype(jnp.float32))
    v = jnp.stack([vx, vy, vz], axis=-1)
    f_vs = jax.vmap(lambda vv, ff: vv[ff])(v, faces)
    x0 = f_vs[..., 0, 0]; y0 = f_vs[..., 0, 1]; z0 = f_vs[..., 0, 2]
    x1 = f_vs[..., 1, 0]; y1 = f_vs[..., 1, 1]; z1 = f_vs[..., 1, 2]
    x2 = f_vs[..., 2, 0]; y2 = f_vs[..., 2, 1]; z2 = f_vs[..., 2, 2]
    denom = (y1 - y2) * (x0 - x2) + (x2 - x1) * (y0 - y2)
    denom = jnp.where(jnp.abs(denom) < 1e-8, 1e-8, denom)
    rd = 1.0 / denom
    a0 = (y1 - y2) * rd
    b0 = (x2 - x1) * rd
    c0 = -(a0 * x2) - b0 * y2
    a1 = (y2 - y0) * rd
    b1 = (x0 - x2) * rd
    c1 = -(a1 * x2) - b1 * y2
    coef = jnp.stack([a0, b0, c0, a1, b1, c1, z0, z1, z2], axis=1)
    bz, _, F = coef.shape
    FP = ((F + FCHUNK - 1) // FCHUNK) * FCHUNK
    if FP != F:
        pad = jnp.zeros((bz, 9, FP - F), jnp.float32)
        pad = pad.at[:, 2, :].set(-1.0)
        coef = jnp.concatenate([coef, pad], axis=2)
    return coef


def _raster_body(coef_ref, tri_ref, depth_acc, idx_acc):
    c = pl.program_id(2)
    row0 = pl.program_id(1) * 8
    px = jax.lax.broadcasted_iota(jnp.int32, (8, W), 1).astype(jnp.float32)
    py = (jax.lax.broadcasted_iota(jnp.int32, (8, W), 0).astype(jnp.float32)
          + jnp.float32(row0))

    @pl.when(c == 0)
    def _init():
        depth_acc[...] = jnp.full((8, W), 1e6, jnp.float32)
        idx_acc[...] = jnp.zeros((8, W), jnp.int32)

    def body(i, carry):
        depth, idx = carry
        a0 = coef_ref[0, 0, i]
        b0 = coef_ref[0, 1, i]
        c0 = coef_ref[0, 2, i]
        a1 = coef_ref[0, 3, i]
        b1 = coef_ref[0, 4, i]
        c1 = coef_ref[0, 5, i]
        z0 = coef_ref[0, 6, i]
        z1 = coef_ref[0, 7, i]
        z2 = coef_ref[0, 8, i]
        w0 = a0 * px + (b0 * py + c0)
        w1 = a1 * px + (b1 * py + c1)
        w2 = (1.0 - w0) - w1
        inside = (w0 >= 0.0) & (w1 >= 0.0) & (w2 >= 0.0)
        d = w0 * z0 + w1 * z1 + w2 * z2
        upd = inside & (d < depth)
        depth = jnp.where(upd, d, depth)
        idx = jnp.where(upd, c * FCHUNK + i, idx)
        return depth, idx

    depth, idx = jax.lax.fori_loop(
        0, FCHUNK, body, (depth_acc[...], idx_acc[...]))
    depth_acc[...] = depth
    idx_acc[...] = idx

    @pl.when(c == pl.num_programs(2) - 1)
    def _fin():
        tri_ref[...] = jnp.where(depth < 1e6, idx, -1)[None]


def _rasterize(coef):
    bz = coef.shape[0]
    FP = coef.shape[2]
    grid = (bz, H // 8, FP // FCHUNK)
    return pl.pallas_call(
        _raster_body,
        grid=grid,
        in_specs=[pl.BlockSpec((1, 9, FCHUNK), lambda b, t, c: (b, 0, c),
                               memory_space=pltpu.SMEM)],
        out_specs=pl.BlockSpec((1, 8, W), lambda b, t, c: (b, t, 0)),
        out_shape=jax.ShapeDtypeStruct((bz, H, W), jnp.int32),
        scratch_shapes=[pltpu.VMEM((8, W), jnp.float32),
                        pltpu.VMEM((8, W), jnp.int32)],
        compiler_params=pltpu.CompilerParams(
            dimension_semantics=("arbitrary", "arbitrary", "arbitrary")),
    )(coef)


def _interp(tri, coef, attributes):
    bz, F = attributes.shape[0], attributes.shape[1]
    D = attributes.shape[-1]
    ys, xs = jnp.meshgrid(jnp.arange(H, dtype=jnp.float32),
                          jnp.arange(W, dtype=jnp.float32), indexing='ij')
    px = xs.reshape(-1)[None]
    py = ys.reshape(-1)[None]
    t = tri.reshape(bz, -1)
    valid = t >= 0
    tc = jnp.where(valid, t, 0)

    def per_batch(coef_b, tc_b):
        g = coef_b[:, tc_b]
        return g
    g = jax.vmap(per_batch)(coef, tc)
    w0 = g[:, 0] * px + (g[:, 1] * py + g[:, 2])
    w1 = g[:, 3] * px + (g[:, 4] * py + g[:, 5])
    w2 = (1.0 - w0) - w1
    attrs = attributes.reshape(bz, F, 3 * D)
    av = jax.vmap(lambda a_b, t_b: a_b[t_b])(attrs, tc)
    av = av.reshape(bz, -1, 3, D)
    pix = (w0[..., None] * av[:, :, 0] + w1[..., None] * av[:, :, 1]
           + w2[..., None] * av[:, :, 2])
    pix = jnp.where(valid[..., None], pix, 0.0)
    vis = valid.astype(jnp.float32)
    out = jnp.concatenate([pix, vis[..., None]], axis=-1)
    out = out.reshape(bz, H, W, D + 1)
    return jnp.transpose(out, (0, 3, 1, 2))


def kernel(vertices, faces, attributes):
    faces = faces.astype(jnp.int32)
    coef = _face_coeffs(vertices, faces)
    tri = _rasterize(coef)
    return _interp(tri, coef, attributes)


# speedup vs baseline: 1.2816x; 1.2816x over previous
import functools

import jax
import jax.numpy as jnp
from jax.experimental import pallas as pl
from jax.experimental.pallas import tpu as pltpu

H = 128
W = 128
FCHUNK = 512


def _transform_vertices(v):
    h, w = H, W
    vx = -v[..., 0]
    vy = -v[..., 1]
    vz = v[..., 2]
    vx = vx * w / 2 + w / 2
    vy = vy * h / 2 + h / 2
    vx = w - 1 - vx
    vy = h - 1 - vy
    vx = -1 + (2 * vx + 1) / w
    vy = -1 + (2 * vy + 1) / h
    vx = vx * w / 2 + w / 2
    vy = vy * h / 2 + h / 2
    vz = vz * w / 2
    return vx, vy, vz


def _face_coeffs(vertices, faces):
    vx, vy, vz = _transform_vertices(vertices.astype(jnp.float32))
    v = jnp.stack([vx, vy, vz], axis=-1)
    f_vs = jax.vmap(lambda vv, ff: vv[ff])(v, faces)
    x0 = f_vs[..., 0, 0]; y0 = f_vs[..., 0, 1]; z0 = f_vs[..., 0, 2]
    x1 = f_vs[..., 1, 0]; y1 = f_vs[..., 1, 1]; z1 = f_vs[..., 1, 2]
    x2 = f_vs[..., 2, 0]; y2 = f_vs[..., 2, 1]; z2 = f_vs[..., 2, 2]
    denom = (y1 - y2) * (x0 - x2) + (x2 - x1) * (y0 - y2)
    denom = jnp.where(jnp.abs(denom) < 1e-8, 1e-8, denom)
    rd = 1.0 / denom
    a0 = (y1 - y2) * rd
    b0 = (x2 - x1) * rd
    c0 = -(a0 * x2) - b0 * y2
    a1 = (y2 - y0) * rd
    b1 = (x0 - x2) * rd
    c1 = -(a1 * x2) - b1 * y2
    coef = jnp.stack([a0, b0, c0, a1, b1, c1, z0, z1, z2], axis=1)
    bz, _, F = coef.shape
    FP = ((F + FCHUNK - 1) // FCHUNK) * FCHUNK
    if FP != F:
        pad = jnp.zeros((bz, 9, FP - F), jnp.float32)
        pad = pad.at[:, 2, :].set(-1.0)
        coef = jnp.concatenate([coef, pad], axis=2)
    return coef


def _raster_body(coef_ref, tri_ref, depth_acc, idx_acc):
    c = pl.program_id(2)
    row0 = pl.program_id(1) * 8
    px = jax.lax.broadcasted_iota(jnp.int32, (8, W), 1).astype(jnp.float32)
    py = (jax.lax.broadcasted_iota(jnp.int32, (8, W), 0).astype(jnp.float32)
          + jnp.float32(row0))

    @pl.when(c == 0)
    def _init():
        depth_acc[...] = jnp.full((8, W), 1e6, jnp.float32)
        idx_acc[...] = jnp.zeros((8, W), jnp.int32)

    def body(i, carry):
        depth, idx = carry
        a0 = coef_ref[0, 0, i]
        b0 = coef_ref[0, 1, i]
        c0 = coef_ref[0, 2, i]
        a1 = coef_ref[0, 3, i]
        b1 = coef_ref[0, 4, i]
        c1 = coef_ref[0, 5, i]
        z0 = coef_ref[0, 6, i]
        z1 = coef_ref[0, 7, i]
        z2 = coef_ref[0, 8, i]
        w0 = a0 * px + (b0 * py + c0)
        w1 = a1 * px + (b1 * py + c1)
        w2 = (1.0 - w0) - w1
        inside = (w0 >= 0.0) & (w1 >= 0.0) & (w2 >= 0.0)
        d = w0 * z0 + w1 * z1 + w2 * z2
        upd = inside & (d < depth)
        depth = jnp.where(upd, d, depth)
        idx = jnp.where(upd, c * FCHUNK + i, idx)
        return depth, idx

    depth, idx = jax.lax.fori_loop(
        0, FCHUNK, body, (depth_acc[...], idx_acc[...]), unroll=16)
    depth_acc[...] = depth
    idx_acc[...] = idx

    @pl.when(c == pl.num_programs(2) - 1)
    def _fin():
        tri_ref[...] = jnp.where(depth < 1e6, idx, -1)[None]


def _rasterize(coef):
    bz = coef.shape[0]
    FP = coef.shape[2]
    grid = (bz, H // 8, FP // FCHUNK)
    return pl.pallas_call(
        _raster_body,
        grid=grid,
        in_specs=[pl.BlockSpec((1, 9, FCHUNK), lambda b, t, c: (b, 0, c),
                               memory_space=pltpu.SMEM)],
        out_specs=pl.BlockSpec((1, 8, W), lambda b, t, c: (b, t, 0)),
        out_shape=jax.ShapeDtypeStruct((bz, H, W), jnp.int32),
        scratch_shapes=[pltpu.VMEM((8, W), jnp.float32),
                        pltpu.VMEM((8, W), jnp.int32)],
        compiler_params=pltpu.CompilerParams(
            dimension_semantics=("arbitrary", "arbitrary", "arbitrary")),
    )(coef)


def _interp(tri, coef, attributes):
    bz, F = attributes.shape[0], attributes.shape[1]
    D = attributes.shape[-1]
    ys, xs = jnp.meshgrid(jnp.arange(H, dtype=jnp.float32),
                          jnp.arange(W, dtype=jnp.float32), indexing='ij')
    px = xs.reshape(-1)[None]
    py = ys.reshape(-1)[None]
    t = tri.reshape(bz, -1)
    valid = t >= 0
    tc = jnp.where(valid, t, 0)

    def per_batch(coef_b, tc_b):
        g = coef_b[:, tc_b]
        return g
    g = jax.vmap(per_batch)(coef, tc)
    w0 = g[:, 0] * px + (g[:, 1] * py + g[:, 2])
    w1 = g[:, 3] * px + (g[:, 4] * py + g[:, 5])
    w2 = (1.0 - w0) - w1
    attrs = attributes.reshape(bz, F, 3 * D)
    av = jax.vmap(lambda a_b, t_b: a_b[t_b])(attrs, tc)
    av = av.reshape(bz, -1, 3, D)
    pix = (w0[..., None] * av[:, :, 0] + w1[..., None] * av[:, :, 1]
           + w2[..., None] * av[:, :, 2])
    pix = jnp.where(valid[..., None], pix, 0.0)
    vis = valid.astype(jnp.float32)
    out = jnp.concatenate([pix, vis[..., None]], axis=-1)
    out = out.reshape(bz, H, W, D + 1)
    return jnp.transpose(out, (0, 3, 1, 2))


def kernel(vertices, faces, attributes):
    faces = faces.astype(jnp.int32)
    coef = _face_coeffs(vertices, faces)
    tri = _rasterize(coef)
    return _interp(tri, coef, attributes)
